# baseline (speedup 1.0000x reference)
import functools

import jax
import jax.numpy as jnp
import numpy as np
from jax import lax
from jax.experimental import pallas as pl
from jax.experimental.pallas import tpu as pltpu
from jax.experimental.pallas import tpu_sc as plsc

NC = 2
NS = 16
NW = NC * NS
LANES = 16

HID = 64
NQ = HID // LANES

C = 128
NBUF = 2

_MAGIC = np.int32(0x5F3759DF)


def _renorm_rows(src, dst, n_rows):
    def body(r, carry):
        qs = [src[r, pl.ds(k * LANES, LANES)] for k in range(NQ)]
        ssv = qs[0] * qs[0]
        for q in qs[1:]:
            ssv = ssv + q * q
        ss = jnp.sum(ssv)
        ssb = jnp.full((LANES,), ss, jnp.float32)
        bits = plsc.bitcast(ssb, jnp.int32)
        y = plsc.bitcast(_MAGIC - (bits >> 1), jnp.float32)
        h = ssb * jnp.float32(0.5)
        for _ in range(3):
            y = y * (jnp.float32(1.5) - h * y * y)
        scale = jnp.where(ssb > jnp.float32(1.0), y, jnp.float32(1.0))
        for k in range(NQ):
            dst[r, pl.ds(k * LANES, LANES)] = qs[k] * scale
        return carry

    lax.fori_loop(0, n_rows, body, 0)


def _encoder_body(n_rows, n_spk, idx_hbm, spk_hbm, lut_p_hbm, lut_s_hbm,
                  out_hbm, ident_hbm, idx_v, sidx_v, gbuf, obuf, *sems):
    gsems = sems[:NBUF]
    osems = sems[NBUF:]
    rpw = n_rows // NW
    spw = n_spk // NW
    nstep = rpw // C

    wid = lax.axis_index("s") * NC + lax.axis_index("c")
    base = wid * rpw

    pltpu.sync_copy(idx_hbm.at[pl.ds(base, rpw)], idx_v)

    for b in range(NBUF):
        pltpu.async_copy(lut_p_hbm.at[idx_v.at[pl.ds(b * C, C)]],
                         gbuf.at[b], gsems[b])

    @pl.loop(0, nstep, step=NBUF)
    def _step(s0):
        for b in range(NBUF):
            s = s0 + b
            pltpu.make_async_copy(lut_p_hbm.at[idx_v.at[pl.ds(s * C, C)]],
                                  gbuf.at[b], gsems[b]).wait()

            @pl.when(s0 >= NBUF)
            def _():
                pltpu.make_async_copy(
                    obuf.at[b],
                    out_hbm.at[pl.ds(base + (s - NBUF) * C, C)],
                    osems[b]).wait()

            _renorm_rows(gbuf.at[b], obuf.at[b], C)

            pltpu.async_copy(obuf.at[b],
                             out_hbm.at[pl.ds(base + s * C, C)], osems[b])

            @pl.when(s0 + NBUF < nstep)
            def _():
                pltpu.async_copy(
                    lut_p_hbm.at[idx_v.at[pl.ds((s + NBUF) * C, C)]],
                    gbuf.at[b], gsems[b])

    for b in range(NBUF):
        pltpu.make_async_copy(
            obuf.at[b],
            out_hbm.at[pl.ds(base + (nstep - NBUF + b) * C, C)],
            osems[b]).wait()

    sbase = wid * spw
    pltpu.sync_copy(spk_hbm.at[pl.ds(sbase, spw)], sidx_v)
    pltpu.async_copy(lut_s_hbm.at[sidx_v], gbuf.at[0], gsems[0]).wait()
    _renorm_rows(gbuf.at[0], obuf.at[0], spw)
    pltpu.sync_copy(obuf.at[0].at[pl.ds(0, spw)],
                    ident_hbm.at[pl.ds(sbase, spw)])


@functools.partial(jax.jit, static_argnums=(4, 5))
def _encoder(idx, spk, lut_p, lut_s, n_rows, n_spk):
    rpw = n_rows // NW
    grid_kernel = functools.partial(
        pl.kernel,
        out_type=[
            jax.ShapeDtypeStruct((n_rows, HID), jnp.float32),
            jax.ShapeDtypeStruct((n_spk, HID), jnp.float32),
        ],
        mesh=plsc.VectorSubcoreMesh(core_axis_name="c", subcore_axis_name="s",
                                    num_cores=NC, num_subcores=NS),
        compiler_params=pltpu.CompilerParams(needs_layout_passes=False,
                                             use_tc_tiling_on_sc=False),
        scratch_types=[
            pltpu.VMEM((rpw,), jnp.int32),
            pltpu.VMEM((n_spk // NW,), jnp.int32),
            pltpu.VMEM((NBUF, C, HID), jnp.float32),
            pltpu.VMEM((NBUF, C, HID), jnp.float32),
        ] + [pltpu.SemaphoreType.DMA] * (2 * NBUF),
    )
    body = functools.partial(_encoder_body, n_rows, n_spk)
    return grid_kernel(body)(idx, spk, lut_p, lut_s)


def kernel(input, speakers, lut_p, lut_s):
    b, l = input.shape
    idx = input.reshape(-1).astype(jnp.int32)
    spk = speakers.astype(jnp.int32)
    outputs, ident = _encoder(idx, spk, lut_p, lut_s, b * l, speakers.shape[0])
    return outputs.reshape(b, l, HID), ident


# speedup vs baseline: 1.0273x; 1.0273x over previous
import functools

import jax
import jax.numpy as jnp
import numpy as np
from jax import lax
from jax.experimental import pallas as pl
from jax.experimental.pallas import tpu as pltpu
from jax.experimental.pallas import tpu_sc as plsc

NC = 2
NS = 16
NW = NC * NS
LANES = 16

HID = 64
NQ = HID // LANES

C = 200
SPW_C = 128
NBUF = 2
G = 8

_MAGIC = np.int32(0x5F3759DF)


def _renorm_rows(src, dst, n_rows):
    lanes = lax.broadcasted_iota(jnp.int32, (LANES,), 0)

    def body(g, carry):
        r0 = g * G
        qs_rows = []
        ssb = jnp.zeros((LANES,), jnp.float32)
        for u in range(G):
            r = r0 + u
            qs = [src[r, pl.ds(k * LANES, LANES)] for k in range(NQ)]
            ssv = qs[0] * qs[0]
            for q in qs[1:]:
                ssv = ssv + q * q
            ssb = jnp.where(lanes == u, jnp.full((LANES,), jnp.sum(ssv)), ssb)
            qs_rows.append(qs)
        bits = plsc.bitcast(ssb, jnp.int32)
        y = plsc.bitcast(_MAGIC - (bits >> 1), jnp.float32)
        h = ssb * jnp.float32(0.5)
        for _ in range(3):
            y = y * (jnp.float32(1.5) - h * y * y)
        scale = jnp.where(ssb > jnp.float32(1.0), y, jnp.float32(1.0))
        for u in range(G):
            r = r0 + u
            sb = scale.at[jnp.full((LANES,), u, jnp.int32)].get(
                mode="promise_in_bounds")
            for k in range(NQ):
                dst[r, pl.ds(k * LANES, LANES)] = qs_rows[u][k] * sb
        return carry

    lax.fori_loop(0, n_rows // G, body, 0)


def _encoder_body(n_rows, n_spk, idx_hbm, spk_hbm, lut_p_hbm, lut_s_hbm,
                  out_hbm, ident_hbm, idx_v, sidx_v, gbuf, obuf, *sems):
    gsems = sems[:NBUF]
    osems = sems[NBUF:]
    rpw = n_rows // NW
    spw = n_spk // NW
    nstep = rpw // C

    wid = lax.axis_index("s") * NC + lax.axis_index("c")
    base = wid * rpw
    b0 = wid * (rpw // C)

    pltpu.sync_copy(idx_hbm.at[pl.ds(base, rpw)], idx_v)

    for b in range(NBUF):
        pltpu.async_copy(lut_p_hbm.at[idx_v.at[pl.ds(b * C, C)]],
                         gbuf.at[b], gsems[b])

    @pl.loop(0, nstep, step=NBUF)
    def _step(s0):
        for b in range(NBUF):
            s = s0 + b
            pltpu.make_async_copy(lut_p_hbm.at[idx_v.at[pl.ds(s * C, C)]],
                                  gbuf.at[b], gsems[b]).wait()

            @pl.when(s0 >= NBUF)
            def _():
                pltpu.make_async_copy(
                    obuf.at[b], out_hbm.at[b0 + s - NBUF], osems[b]).wait()

            _renorm_rows(gbuf.at[b], obuf.at[b], C)

            pltpu.async_copy(obuf.at[b], out_hbm.at[b0 + s], osems[b])

            @pl.when(s0 + NBUF < nstep)
            def _():
                pltpu.async_copy(
                    lut_p_hbm.at[idx_v.at[pl.ds((s + NBUF) * C, C)]],
                    gbuf.at[b], gsems[b])

    for b in range(NBUF):
        pltpu.make_async_copy(
            obuf.at[b], out_hbm.at[b0 + nstep - NBUF + b], osems[b]).wait()

    sbase = wid * spw
    pltpu.sync_copy(spk_hbm.at[pl.ds(sbase, spw)], sidx_v)
    pltpu.async_copy(lut_s_hbm.at[sidx_v], gbuf.at[0].at[pl.ds(0, spw)],
                     gsems[0]).wait()
    _renorm_rows(gbuf.at[0], obuf.at[0], spw)
    pltpu.sync_copy(obuf.at[0].at[pl.ds(0, spw)],
                    ident_hbm.at[pl.ds(sbase, spw)])


@functools.partial(jax.jit, static_argnums=(4, 5, 6))
def _encoder(idx, spk, lut_p, lut_s, n_batch, n_len, n_spk):
    n_rows = n_batch * n_len
    rpw = n_rows // NW
    grid_kernel = functools.partial(
        pl.kernel,
        out_type=[
            jax.ShapeDtypeStruct((n_batch, n_len, HID), jnp.float32),
            jax.ShapeDtypeStruct((n_spk, HID), jnp.float32),
        ],
        mesh=plsc.VectorSubcoreMesh(core_axis_name="c", subcore_axis_name="s",
                                    num_cores=NC, num_subcores=NS),
        compiler_params=pltpu.CompilerParams(needs_layout_passes=False,
                                             use_tc_tiling_on_sc=False),
        scratch_types=[
            pltpu.VMEM((rpw,), jnp.int32),
            pltpu.VMEM((n_spk // NW,), jnp.int32),
            pltpu.VMEM((NBUF, C, HID), jnp.float32),
            pltpu.VMEM((NBUF, C, HID), jnp.float32),
        ] + [pltpu.SemaphoreType.DMA] * (2 * NBUF),
    )
    body = functools.partial(_encoder_body, n_rows, n_spk)
    return grid_kernel(body)(idx, spk, lut_p, lut_s)


def kernel(input, speakers, lut_p, lut_s):
    b, l = input.shape
    idx = input.reshape(-1).astype(jnp.int32)
    spk = speakers.astype(jnp.int32)
    outputs, ident = _encoder(idx, spk, lut_p, lut_s, b, l, speakers.shape[0])
    return outputs, ident


# speedup vs baseline: 1.1589x; 1.1281x over previous
import functools

import jax
import jax.numpy as jnp
import numpy as np
from jax import lax
from jax.experimental import pallas as pl
from jax.experimental.pallas import tpu as pltpu
from jax.experimental.pallas import tpu_sc as plsc

NC = 2
NS = 16
NW = NC * NS
LANES = 16

HID = 64
PADW = 128
NQ = HID // LANES

C = 200
SPW_C = 128
NBUF = 2
G = 8

_MAGIC = np.int32(0x5F3759DF)


def _renorm_rows(src, dst, n_rows):
    lanes = lax.broadcasted_iota(jnp.int32, (LANES,), 0)

    def body(g, carry):
        r0 = g * G
        qs_rows = []
        ssb = jnp.zeros((LANES,), jnp.float32)
        for u in range(G):
            r = r0 + u
            qs = [src[r, pl.ds(k * LANES, LANES)] for k in range(NQ)]
            ssv = qs[0] * qs[0]
            for q in qs[1:]:
                ssv = ssv + q * q
            ssb = jnp.where(lanes == u, jnp.full((LANES,), jnp.sum(ssv)), ssb)
            qs_rows.append(qs)
        bits = plsc.bitcast(ssb, jnp.int32)
        y = plsc.bitcast(_MAGIC - (bits >> 1), jnp.float32)
        h = ssb * jnp.float32(0.5)
        for _ in range(3):
            y = y * (jnp.float32(1.5) - h * y * y)
        scale = jnp.where(ssb > jnp.float32(1.0), y, jnp.float32(1.0))
        for u in range(G):
            r = r0 + u
            sb = scale.at[jnp.full((LANES,), u, jnp.int32)].get(
                mode="promise_in_bounds")
            for k in range(NQ):
                dst[r, pl.ds(k * LANES, LANES)] = qs_rows[u][k] * sb
        return carry

    lax.fori_loop(0, n_rows // G, body, 0)


def _encoder_body(n_rows, n_spk, idx_hbm, spk_hbm, lut_p_hbm, lut_s_hbm,
                  out_hbm, ident_hbm, idx_v, sidx_v, gbuf, obuf, *sems):
    gsems = sems[:NBUF]
    osems = sems[NBUF:]
    rpw = n_rows // NW
    spw = n_spk // NW
    nstep = rpw // C

    wid = lax.axis_index("s") * NC + lax.axis_index("c")
    base = wid * rpw
    b0 = wid * (rpw // C)

    pltpu.sync_copy(idx_hbm.at[pl.ds(base, rpw)], idx_v)

    for b in range(NBUF):
        pltpu.async_copy(lut_p_hbm.at[idx_v.at[pl.ds(b * C, C)]],
                         gbuf.at[b], gsems[b])

    @pl.loop(0, nstep, step=NBUF)
    def _step(s0):
        for b in range(NBUF):
            s = s0 + b
            pltpu.make_async_copy(lut_p_hbm.at[idx_v.at[pl.ds(s * C, C)]],
                                  gbuf.at[b], gsems[b]).wait()

            @pl.when(s0 >= NBUF)
            def _():
                pltpu.make_async_copy(
                    obuf.at[b], out_hbm.at[b0 + s - NBUF], osems[b]).wait()

            _renorm_rows(gbuf.at[b], obuf.at[b], C)

            pltpu.async_copy(obuf.at[b], out_hbm.at[b0 + s], osems[b])

            @pl.when(s0 + NBUF < nstep)
            def _():
                pltpu.async_copy(
                    lut_p_hbm.at[idx_v.at[pl.ds((s + NBUF) * C, C)]],
                    gbuf.at[b], gsems[b])

    for b in range(NBUF):
        pltpu.make_async_copy(
            obuf.at[b], out_hbm.at[b0 + nstep - NBUF + b], osems[b]).wait()

    sbase = wid * spw
    pltpu.sync_copy(spk_hbm.at[pl.ds(sbase, spw)], sidx_v)
    pltpu.async_copy(lut_s_hbm.at[sidx_v], gbuf.at[0].at[pl.ds(0, spw)],
                     gsems[0]).wait()
    _renorm_rows(gbuf.at[0], obuf.at[0], spw)
    pltpu.sync_copy(obuf.at[0].at[pl.ds(0, spw)],
                    ident_hbm.at[pl.ds(sbase, spw)])


@functools.partial(jax.jit, static_argnums=(4, 5, 6))
def _encoder(idx, spk, lut_p, lut_s, n_batch, n_len, n_spk):
    n_rows = n_batch * n_len
    rpw = n_rows // NW
    grid_kernel = functools.partial(
        pl.kernel,
        out_type=[
            jax.ShapeDtypeStruct((n_batch, n_len, HID), jnp.float32),
            jax.ShapeDtypeStruct((n_spk, HID), jnp.float32),
        ],
        mesh=plsc.VectorSubcoreMesh(core_axis_name="c", subcore_axis_name="s",
                                    num_cores=NC, num_subcores=NS),
        compiler_params=pltpu.CompilerParams(needs_layout_passes=False,
                                             use_tc_tiling_on_sc=True),
        scratch_types=[
            pltpu.VMEM((rpw,), jnp.int32),
            pltpu.VMEM((n_spk // NW,), jnp.int32),
            pltpu.VMEM((NBUF, C, PADW), jnp.float32),
            pltpu.VMEM((NBUF, C, HID), jnp.float32),
        ] + [pltpu.SemaphoreType.DMA] * (2 * NBUF),
    )
    body = functools.partial(_encoder_body, n_rows, n_spk)
    return grid_kernel(body)(idx, spk, lut_p, lut_s)


def kernel(input, speakers, lut_p, lut_s):
    b, l = input.shape
    idx = input.reshape(-1).astype(jnp.int32)
    spk = speakers.astype(jnp.int32)
    lut_p_pad = jnp.pad(lut_p, ((0, 0), (0, PADW - HID)))
    lut_s_pad = jnp.pad(lut_s, ((0, 0), (0, PADW - HID)))
    outputs, ident = _encoder(idx, spk, lut_p_pad, lut_s_pad, b, l,
                              speakers.shape[0])
    return outputs, ident
